# Initial kernel scaffold; baseline (speedup 1.0000x reference)
import jax
import jax.numpy as jnp
from jax.experimental import pallas as pl


def kernel(x0, x1, x2, up_index0, up_shared0, up_index1, up_shared1, down_index1, down_shared1, down_index2, down_shared2, batch0, batch1, batch2, Wup, bup, gup, beup, Wdn, bdn, gdn, bedn, Wu1, bu1, Wu2, bu2, gu, beu, Wl1, bl1, Wl2, bl2):
    raise NotImplementedError("write your pallas kernel here")


# speedup vs baseline: 3.3953x; 3.3953x over previous
import functools

import jax
import jax.numpy as jnp
from jax import lax
from jax.experimental import pallas as pl
from jax.experimental.pallas import tpu as pltpu

D = 128
NB = 32


def _premult_body(nt, x_ref, w_ref, r_ref, *o_refs):
    res = jnp.dot(x_ref[...], w_ref[...], preferred_element_type=jnp.float32)
    res = res + r_ref[...]
    for j in range(nt):
        o_refs[j][...] = res[:, j * D:(j + 1) * D]


def _premult(x, w, rowc, blk):
    n = x.shape[0]
    k = w.shape[1]
    nt = k // D
    return pl.pallas_call(
        functools.partial(_premult_body, nt),
        grid=(n // blk,),
        in_specs=[
            pl.BlockSpec((blk, D), lambda i: (i, 0)),
            pl.BlockSpec((D, k), lambda i: (0, 0)),
            pl.BlockSpec((1, k), lambda i: (0, 0)),
        ],
        out_specs=[pl.BlockSpec((blk, D), lambda i: (i, 0))] * nt,
        out_shape=[jax.ShapeDtypeStruct((n, D), jnp.float32)] * nt,
    )(x, w, rowc)


def _upd_body2(x_ref, su_ref, cu_ref, sd_ref, cd_ref, aff_ref, w1_ref, w2_ref,
               h_ref, st_ref):
    i = pl.program_id(0)
    aff = aff_ref[...]
    u = (x_ref[...] * aff[0] + aff[1]
         + su_ref[...] * aff[2] + cu_ref[...] * aff[3]
         + sd_ref[...] * aff[4] + cd_ref[...] * aff[5])
    h1 = jnp.maximum(jnp.dot(u, w1_ref[...], preferred_element_type=jnp.float32) + aff[6], 0.0)
    h2 = jnp.maximum(jnp.dot(h1, w2_ref[...], preferred_element_type=jnp.float32) + aff[7], 0.0)
    h_ref[...] = h2

    @pl.when(i == 0)
    def _():
        st_ref[...] = jnp.zeros_like(st_ref)

    s1 = jnp.sum(h2, axis=0)
    s2 = jnp.sum(h2 * h2, axis=0)
    st_ref[...] += jnp.concatenate(
        [s1[None], s2[None], jnp.zeros((6, D), jnp.float32)], axis=0)


def _upd_body1(x_ref, su_ref, cu_ref, aff_ref, w1_ref, w2_ref, h_ref, st_ref):
    i = pl.program_id(0)
    aff = aff_ref[...]
    u = (x_ref[...] * aff[0] + aff[1]
         + su_ref[...] * aff[2] + cu_ref[...] * aff[3])
    h1 = jnp.maximum(jnp.dot(u, w1_ref[...], preferred_element_type=jnp.float32) + aff[6], 0.0)
    h2 = jnp.maximum(jnp.dot(h1, w2_ref[...], preferred_element_type=jnp.float32) + aff[7], 0.0)
    h_ref[...] = h2

    @pl.when(i == 0)
    def _():
        st_ref[...] = jnp.zeros_like(st_ref)

    s1 = jnp.sum(h2, axis=0)
    s2 = jnp.sum(h2 * h2, axis=0)
    st_ref[...] += jnp.concatenate(
        [s1[None], s2[None], jnp.zeros((6, D), jnp.float32)], axis=0)


def _update(x, su, cu, sd, cd, aff, w1, w2, blk):
    n = x.shape[0]
    two = sd is not None
    body = _upd_body2 if two else _upd_body1
    in_arrays = [x, su, cu] + ([sd, cd] if two else []) + [aff, w1, w2]
    in_specs = ([pl.BlockSpec((blk, D), lambda i: (i, 0)),
                 pl.BlockSpec((blk, D), lambda i: (i, 0)),
                 pl.BlockSpec((blk, 1), lambda i: (i, 0))]
                + ([pl.BlockSpec((blk, D), lambda i: (i, 0)),
                    pl.BlockSpec((blk, 1), lambda i: (i, 0))] if two else [])
                + [pl.BlockSpec((8, D), lambda i: (0, 0)),
                   pl.BlockSpec((D, D), lambda i: (0, 0)),
                   pl.BlockSpec((D, D), lambda i: (0, 0))])
    return pl.pallas_call(
        body,
        grid=(n // blk,),
        in_specs=in_specs,
        out_specs=[pl.BlockSpec((blk, D), lambda i: (i, 0)),
                   pl.BlockSpec((8, D), lambda i: (0, 0))],
        out_shape=[jax.ShapeDtypeStruct((n, D), jnp.float32),
                   jax.ShapeDtypeStruct((8, D), jnp.float32)],
    )(*in_arrays)


def _pool_body(h_ref, b_ref, aff_ref, ps_ref, pc_ref):
    i = pl.program_id(0)

    @pl.when(i == 0)
    def _():
        ps_ref[...] = jnp.zeros_like(ps_ref)
        pc_ref[...] = jnp.zeros_like(pc_ref)

    aff = aff_ref[...]
    xnew = h_ref[...] * aff[0] + aff[1]
    bid = b_ref[...]
    cols = lax.broadcasted_iota(jnp.int32, (bid.shape[0], NB), 1)
    onehot = (bid == cols).astype(jnp.float32)
    ps_ref[...] += lax.dot_general(onehot, xnew, (((0,), (0,)), ((), ())),
                                   preferred_element_type=jnp.float32)
    pc_ref[...] += jnp.sum(onehot, axis=0)[:, None]


def _pool(h, batch, aff, blk):
    n = h.shape[0]
    return pl.pallas_call(
        _pool_body,
        grid=(n // blk,),
        in_specs=[pl.BlockSpec((blk, D), lambda i: (i, 0)),
                  pl.BlockSpec((blk, 1), lambda i: (i, 0)),
                  pl.BlockSpec((8, D), lambda i: (0, 0))],
        out_specs=[pl.BlockSpec((NB, D), lambda i: (0, 0)),
                   pl.BlockSpec((NB, D), lambda i: (0, 0))],
        out_shape=[jax.ShapeDtypeStruct((NB, D), jnp.float32),
                   jax.ShapeDtypeStruct((NB, D), jnp.float32)],
    )(h, batch.reshape(n, 1), aff)


def _final_body(ps0, pc0, ps1, pc1, ps2, pc2, w1_ref, b1_ref, w2_ref, b2_ref,
                o_ref):
    p = (ps0[...] / jnp.maximum(pc0[...], 1.0)
         + ps1[...] / jnp.maximum(pc1[...], 1.0)
         + ps2[...] / jnp.maximum(pc2[...], 1.0))
    h = jnp.maximum(jnp.dot(p, w1_ref[...], preferred_element_type=jnp.float32)
                    + b1_ref[...], 0.0)
    o = jnp.dot(h, w2_ref[...], preferred_element_type=jnp.float32) + b2_ref[...]
    m = jnp.max(o, axis=-1, keepdims=True)
    lse = m + jnp.log(jnp.sum(jnp.exp(o - m), axis=-1, keepdims=True))
    o_ref[...] = o - lse


def _final(ps0, pc0, ps1, pc1, ps2, pc2, wl1, bl1, wl2, bl2):
    c = wl2.shape[1]
    return pl.pallas_call(
        _final_body,
        out_shape=jax.ShapeDtypeStruct((NB, c), jnp.float32),
    )(ps0, pc0, ps1, pc1, ps2, pc2, wl1, bl1.reshape(1, -1), wl2,
      bl2.reshape(1, -1))


def _edge_jnp(p_tab, q_tab, src, sh, dst, ndst):
    y = jax.nn.relu(p_tab[src] + q_tab[sh])
    e = y.shape[0]
    ssum = y.sum(axis=0)
    ssq = (y * y).sum(axis=0)
    s = jax.ops.segment_sum(y, dst, num_segments=ndst)
    cnt = jax.ops.segment_sum(jnp.ones((e,), jnp.float32), dst,
                              num_segments=ndst)
    return s, cnt, ssum, ssq


def _bn_affine(ssum, ssq, e, g, be):
    m = ssum / e
    v = ssq / e - m * m
    a = g * lax.rsqrt(v + 1e-5)
    c = be - a * m
    return a, c


def _aff8(ax, cx, au, cu, ad, cd, b1, b2):
    return jnp.stack([ax, cx, au, cu, ad, cd, b1, b2], axis=0)


def kernel(x0, x1, x2, up_index0, up_shared0, up_index1, up_shared1,
           down_index1, down_shared1, down_index2, down_shared2,
           batch0, batch1, batch2, Wup, bup, gup, beup, Wdn, bdn, gdn, bedn,
           Wu1, bu1, Wu2, bu2, gu, beu, Wl1, bl1, Wl2, bl2):
    n0, n1, n2 = x0.shape[0], x1.shape[0], x2.shape[0]
    l_layers = Wup.shape[0]
    blk = 2000

    zer = jnp.zeros((D,), jnp.float32)
    one = jnp.ones((D,), jnp.float32)
    an0 = an1 = an2 = one
    cn0 = cn1 = cn2 = zer
    h0, h1, h2x = x0, x1, x2

    for l in range(l_layers):
        wu_t, wu_b = Wup[l][:D], Wup[l][D:]
        wd_t, wd_b = Wdn[l][:D], Wdn[l][D:]
        w0 = jnp.concatenate([wu_t, wd_b], axis=1)
        r0 = (cn0 @ w0 + jnp.concatenate([bup[l], jnp.zeros((D,))]))[None]
        t0 = _premult(h0, an0[:, None] * w0, r0, blk)

        w1c = jnp.concatenate([wu_b, wu_t, wd_t, wd_b], axis=1)
        r1 = (cn1 @ w1c + jnp.concatenate(
            [jnp.zeros((D,)), bup[l], bdn[l], jnp.zeros((D,))]))[None]
        t1 = _premult(h1, an1[:, None] * w1c, r1, blk)

        w2c = jnp.concatenate([wu_b, wd_t], axis=1)
        r2 = (cn2 @ w2c + jnp.concatenate([jnp.zeros((D,)), bdn[l]]))[None]
        t2 = _premult(h2x, an2[:, None] * w2c, r2, blk)

        s_u0, c_u0, sm, sq = _edge_jnp(t0[0], t1[0], up_index0[0], up_shared0,
                                       up_index0[1], n0)
        a_u0, cc_u0 = _bn_affine(sm, sq, up_shared0.shape[0], gup[l], beup[l])
        s_u1, c_u1, sm, sq = _edge_jnp(t1[1], t2[0], up_index1[0], up_shared1,
                                       up_index1[1], n1)
        a_u1, cc_u1 = _bn_affine(sm, sq, up_shared1.shape[0], gup[l], beup[l])
        s_d1, c_d1, sm, sq = _edge_jnp(t1[2], t0[1], down_index1[0],
                                       down_shared1, down_index1[1], n1)
        a_d1, cc_d1 = _bn_affine(sm, sq, down_shared1.shape[0], gdn[l], bedn[l])
        s_d2, c_d2, sm, sq = _edge_jnp(t2[1], t1[3], down_index2[0],
                                       down_shared2, down_index2[1], n2)
        a_d2, cc_d2 = _bn_affine(sm, sq, down_shared2.shape[0], gdn[l], bedn[l])

        h0, st0 = _update(h0, s_u0, c_u0[:, None], None, None,
                          _aff8(an0, cn0, a_u0, cc_u0, zer, zer, bu1[l], bu2[l]),
                          Wu1[l], Wu2[l], blk)
        h1, st1 = _update(h1, s_u1, c_u1[:, None], s_d1, c_d1[:, None],
                          _aff8(an1, cn1, a_u1, cc_u1, a_d1, cc_d1, bu1[l], bu2[l]),
                          Wu1[l], Wu2[l], blk)
        h2x, st2 = _update(h2x, s_d2, c_d2[:, None], None, None,
                           _aff8(an2, cn2, a_d2, cc_d2, zer, zer, bu1[l], bu2[l]),
                           Wu1[l], Wu2[l], blk)

        an0, cn0 = _bn_affine(st0[0], st0[1], float(n0), gu[l], beu[l])
        an1, cn1 = _bn_affine(st1[0], st1[1], float(n1), gu[l], beu[l])
        an2, cn2 = _bn_affine(st2[0], st2[1], float(n2), gu[l], beu[l])

    ps0, pc0 = _pool(h0, batch0, _aff8(an0, cn0, zer, zer, zer, zer, zer, zer), blk)
    ps1, pc1 = _pool(h1, batch1, _aff8(an1, cn1, zer, zer, zer, zer, zer, zer), blk)
    ps2, pc2 = _pool(h2x, batch2, _aff8(an2, cn2, zer, zer, zer, zer, zer, zer), blk)

    return _final(ps0, pc0, ps1, pc1, ps2, pc2, Wl1, bl1, Wl2, bl2)


# speedup vs baseline: 5.4031x; 1.5913x over previous
import functools

import jax
import jax.numpy as jnp
from jax import lax
from jax.experimental import pallas as pl
from jax.experimental.pallas import tpu as pltpu
from jax.experimental.pallas import tpu_sc as plsc

D = 128
NB = 32


def _premult_body(nt, x_ref, w_ref, r_ref, *o_refs):
    res = jnp.dot(x_ref[...], w_ref[...], preferred_element_type=jnp.float32)
    res = res + r_ref[...]
    for j in range(nt):
        o_refs[j][...] = res[:, j * D:(j + 1) * D]


def _premult(x, w, rowc, blk):
    n = x.shape[0]
    k = w.shape[1]
    nt = k // D
    return pl.pallas_call(
        functools.partial(_premult_body, nt),
        grid=(n // blk,),
        in_specs=[
            pl.BlockSpec((blk, D), lambda i: (i, 0)),
            pl.BlockSpec((D, k), lambda i: (0, 0)),
            pl.BlockSpec((1, k), lambda i: (0, 0)),
        ],
        out_specs=[pl.BlockSpec((blk, D), lambda i: (i, 0))] * nt,
        out_shape=[jax.ShapeDtypeStruct((n, D), jnp.float32)] * nt,
    )(x, w, rowc)


def _upd_body2(x_ref, su_ref, cu_ref, sd_ref, cd_ref, aff_ref, w1_ref, w2_ref,
               h_ref, st_ref):
    i = pl.program_id(0)
    aff = aff_ref[...]
    u = (x_ref[...] * aff[0] + aff[1]
         + su_ref[...] * aff[2] + cu_ref[...] * aff[3]
         + sd_ref[...] * aff[4] + cd_ref[...] * aff[5])
    h1 = jnp.maximum(jnp.dot(u, w1_ref[...], preferred_element_type=jnp.float32) + aff[6], 0.0)
    h2 = jnp.maximum(jnp.dot(h1, w2_ref[...], preferred_element_type=jnp.float32) + aff[7], 0.0)
    h_ref[...] = h2

    @pl.when(i == 0)
    def _():
        st_ref[...] = jnp.zeros_like(st_ref)

    s1 = jnp.sum(h2, axis=0)
    s2 = jnp.sum(h2 * h2, axis=0)
    st_ref[...] += jnp.concatenate(
        [s1[None], s2[None], jnp.zeros((6, D), jnp.float32)], axis=0)


def _upd_body1(x_ref, su_ref, cu_ref, aff_ref, w1_ref, w2_ref, h_ref, st_ref):
    i = pl.program_id(0)
    aff = aff_ref[...]
    u = (x_ref[...] * aff[0] + aff[1]
         + su_ref[...] * aff[2] + cu_ref[...] * aff[3])
    h1 = jnp.maximum(jnp.dot(u, w1_ref[...], preferred_element_type=jnp.float32) + aff[6], 0.0)
    h2 = jnp.maximum(jnp.dot(h1, w2_ref[...], preferred_element_type=jnp.float32) + aff[7], 0.0)
    h_ref[...] = h2

    @pl.when(i == 0)
    def _():
        st_ref[...] = jnp.zeros_like(st_ref)

    s1 = jnp.sum(h2, axis=0)
    s2 = jnp.sum(h2 * h2, axis=0)
    st_ref[...] += jnp.concatenate(
        [s1[None], s2[None], jnp.zeros((6, D), jnp.float32)], axis=0)


def _update(x, su, cu, sd, cd, aff, w1, w2, blk):
    n = x.shape[0]
    two = sd is not None
    body = _upd_body2 if two else _upd_body1
    in_arrays = [x, su, cu] + ([sd, cd] if two else []) + [aff, w1, w2]
    in_specs = ([pl.BlockSpec((blk, D), lambda i: (i, 0)),
                 pl.BlockSpec((blk, D), lambda i: (i, 0)),
                 pl.BlockSpec((blk, 1), lambda i: (i, 0))]
                + ([pl.BlockSpec((blk, D), lambda i: (i, 0)),
                    pl.BlockSpec((blk, 1), lambda i: (i, 0))] if two else [])
                + [pl.BlockSpec((8, D), lambda i: (0, 0)),
                   pl.BlockSpec((D, D), lambda i: (0, 0)),
                   pl.BlockSpec((D, D), lambda i: (0, 0))])
    return pl.pallas_call(
        body,
        grid=(n // blk,),
        in_specs=in_specs,
        out_specs=[pl.BlockSpec((blk, D), lambda i: (i, 0)),
                   pl.BlockSpec((8, D), lambda i: (0, 0))],
        out_shape=[jax.ShapeDtypeStruct((n, D), jnp.float32),
                   jax.ShapeDtypeStruct((8, D), jnp.float32)],
    )(*in_arrays)


def _pool_body(h_ref, b_ref, aff_ref, ps_ref, pc_ref):
    i = pl.program_id(0)

    @pl.when(i == 0)
    def _():
        ps_ref[...] = jnp.zeros_like(ps_ref)
        pc_ref[...] = jnp.zeros_like(pc_ref)

    aff = aff_ref[...]
    xnew = h_ref[...] * aff[0] + aff[1]
    bid = b_ref[...]
    cols = lax.broadcasted_iota(jnp.int32, (bid.shape[0], NB), 1)
    onehot = (bid == cols).astype(jnp.float32)
    ps_ref[...] += lax.dot_general(onehot, xnew, (((0,), (0,)), ((), ())),
                                   preferred_element_type=jnp.float32)
    pc_ref[...] += jnp.sum(onehot, axis=0)[:, None]


def _pool(h, batch, aff, blk):
    n = h.shape[0]
    return pl.pallas_call(
        _pool_body,
        grid=(n // blk,),
        in_specs=[pl.BlockSpec((blk, D), lambda i: (i, 0)),
                  pl.BlockSpec((blk, 1), lambda i: (i, 0)),
                  pl.BlockSpec((8, D), lambda i: (0, 0))],
        out_specs=[pl.BlockSpec((NB, D), lambda i: (0, 0)),
                   pl.BlockSpec((NB, D), lambda i: (0, 0))],
        out_shape=[jax.ShapeDtypeStruct((NB, D), jnp.float32),
                   jax.ShapeDtypeStruct((NB, D), jnp.float32)],
    )(h, batch.reshape(n, 1), aff)


def _final_body(ps0, pc0, ps1, pc1, ps2, pc2, w1_ref, b1_ref, w2_ref, b2_ref,
                o_ref):
    p = (ps0[...] / jnp.maximum(pc0[...], 1.0)
         + ps1[...] / jnp.maximum(pc1[...], 1.0)
         + ps2[...] / jnp.maximum(pc2[...], 1.0))
    h = jnp.maximum(jnp.dot(p, w1_ref[...], preferred_element_type=jnp.float32)
                    + b1_ref[...], 0.0)
    o = jnp.dot(h, w2_ref[...], preferred_element_type=jnp.float32) + b2_ref[...]
    m = jnp.max(o, axis=-1, keepdims=True)
    lse = m + jnp.log(jnp.sum(jnp.exp(o - m), axis=-1, keepdims=True))
    o_ref[...] = o - lse


def _final(ps0, pc0, ps1, pc1, ps2, pc2, wl1, bl1, wl2, bl2):
    c = wl2.shape[1]
    return pl.pallas_call(
        _final_body,
        out_shape=jax.ShapeDtypeStruct((NB, c), jnp.float32),
    )(ps0, pc0, ps1, pc1, ps2, pc2, wl1, bl1.reshape(1, -1), wl2,
      bl2.reshape(1, -1))


_R = 6144
_CE = 2048
_G = 256
_CB = 2560
_EALIGN = 16 * _CE


def _edge_sc(ptab, qtab, srcp, shp, dstp, ndst):
    e_pad = srcp.shape[0]
    nbin = -(-ndst // _R)
    nch = e_pad // _EALIGN
    etile = e_pad // 16
    rows = _R // 16

    def body(ptab_h, qtab_h, src_h, sh_h, dst_h, s_h, cnt_h, st_h,
             ssrc, ssh, sdst, csrc, csh, cdst, gsrc, gsh, gdst,
             pbuf, zbuf, zcnt, onesg, stv, acc, cntacc):
        c = lax.axis_index("c")
        s = lax.axis_index("s")
        iota = lax.iota(jnp.int32, 16)
        tbase = s * etile
        zv = jnp.zeros((16,), jnp.float32)

        def z16(r, _):
            for kc in range(8):
                zbuf[r, pl.ds(kc * 16, 16)] = zv
            return 0
        lax.fori_loop(0, 128, z16, 0)

        def zc16(k, _):
            zcnt[pl.ds(k * 16, 16)] = zv
            return 0
        lax.fori_loop(0, rows // 16, zc16, 0)

        def o16(k, _):
            onesg[pl.ds(k * 16, 16)] = zv + 1.0
            return 0
        lax.fori_loop(0, _G // 16, o16, 0)
        for kc in range(8):
            stv[0, pl.ds(kc * 16, 16)] = zv
            stv[1, pl.ds(kc * 16, 16)] = zv

        def process_group(goff, nvalid, full):
            def stage(k, _):
                base = goff + k * 16
                sv = csrc[pl.ds(base, 16)]
                hv = csh[pl.ds(base, 16)]
                dv = cdst[pl.ds(base, 16)]
                if not full:
                    slot = k * 16 + iota
                    valid = slot < nvalid
                    sv = jnp.where(valid, sv, slot & 127)
                    hv = jnp.where(valid, hv, slot & 127)
                    dv = jnp.where(valid, dv, _R + (iota & 15))
                gsrc[pl.ds(k * 16, 16)] = sv
                gsh[pl.ds(k * 16, 16)] = hv
                gdst[pl.ds(k * 16, 16)] = dv
                return 0
            lax.fori_loop(0, _G // 16, stage, 0)
            pltpu.sync_copy(ptab_h.at[gsrc], pbuf)
            pltpu.sync_copy(qtab_h.at[gsh], pbuf, add=True)

            def crow(r, st):
                out = []
                for kc in range(8):
                    y = jnp.maximum(pbuf[r, pl.ds(kc * 16, 16)], 0.0)
                    pbuf[r, pl.ds(kc * 16, 16)] = y
                    if not full:
                        y = jnp.where(r < nvalid, y, 0.0)
                    out.append((st[kc][0] + y, st[kc][1] + y * y))
                return tuple(out)
            st0 = tuple((zv, zv) for _ in range(8))
            st = lax.fori_loop(0, _G, crow, st0)
            for kc in range(8):
                stv[0, pl.ds(kc * 16, 16)] += st[kc][0]
                stv[1, pl.ds(kc * 16, 16)] += st[kc][1]
            pltpu.sync_copy(pbuf, acc.at[gdst], add=True)
            pltpu.sync_copy(onesg, cntacc.at[gdst], add=True)

        def do_bin(i, _):
            b = 2 * i + c
            lo = b * _R
            for j in range(rows // 128):
                pltpu.sync_copy(zbuf, acc.at[pl.ds(s * rows + j * 128, 128)])
            pltpu.sync_copy(zcnt, cntacc.at[pl.ds(s * rows, rows)])
            plsc.subcore_barrier()

            def do_chunk(ch, cur):
                off = tbase + ch * _CE
                pltpu.sync_copy(src_h.at[pl.ds(off, _CE)], ssrc)
                pltpu.sync_copy(sh_h.at[pl.ds(off, _CE)], ssh)
                pltpu.sync_copy(dst_h.at[pl.ds(off, _CE)], sdst)

                def filt(j, cur):
                    dv = sdst[pl.ds(j * 16, 16)]
                    m = (dv >= lo) & (dv < lo + _R)
                    csum = plsc.cumsum(m.astype(jnp.int32))
                    pos = jnp.where(m, cur + csum - 1, _CB + iota)
                    plsc.store_scatter(cdst, [pos], dv - lo)
                    plsc.store_scatter(csrc, [pos], ssrc[pl.ds(j * 16, 16)])
                    plsc.store_scatter(csh, [pos], ssh[pl.ds(j * 16, 16)])
                    return cur + jnp.max(csum)
                cur = lax.fori_loop(0, _CE // 16, filt, cur)
                ngr = cur // _G

                def dog(g, _):
                    process_group(g * _G, _G, True)
                    return 0
                lax.fori_loop(0, ngr, dog, 0)

                def shift(k, _):
                    v1 = csrc[pl.ds(ngr * _G + k * 16, 16)]
                    v2 = csh[pl.ds(ngr * _G + k * 16, 16)]
                    v3 = cdst[pl.ds(ngr * _G + k * 16, 16)]
                    csrc[pl.ds(k * 16, 16)] = v1
                    csh[pl.ds(k * 16, 16)] = v2
                    cdst[pl.ds(k * 16, 16)] = v3
                    return 0
                lax.fori_loop(0, _G // 16, shift, 0)
                return cur - ngr * _G

            rem = lax.fori_loop(0, nch, do_chunk, jnp.int32(0))

            @pl.when(rem > 0)
            def _():
                process_group(0, rem, False)

            plsc.subcore_barrier()
            pltpu.sync_copy(acc.at[pl.ds(s * rows, rows)],
                            s_h.at[pl.ds(b * _R + s * rows, rows)])
            pltpu.sync_copy(cntacc.at[pl.ds(s * rows, rows)],
                            cnt_h.at[pl.ds(b * _R + s * rows, rows)])
            return 0

        nb_me = jnp.where(c == 0, (nbin + 1) // 2, nbin // 2)
        lax.fori_loop(0, nb_me, do_bin, 0)
        pltpu.sync_copy(stv, st_h.at[s * 2 + c])

    mesh = plsc.VectorSubcoreMesh(core_axis_name="c", subcore_axis_name="s",
                                  num_cores=2, num_subcores=16)
    out_type = [jax.ShapeDtypeStruct((nbin * _R, D), jnp.float32),
                jax.ShapeDtypeStruct((nbin * _R,), jnp.float32),
                jax.ShapeDtypeStruct((32, 2, D), jnp.float32)]
    scratch = ([pltpu.VMEM((_CE,), jnp.int32)] * 3
               + [pltpu.VMEM((_CB + 16,), jnp.int32)] * 3
               + [pltpu.VMEM((_G,), jnp.int32)] * 3
               + [pltpu.VMEM((_G, D), jnp.float32),
                  pltpu.VMEM((128, D), jnp.float32),
                  pltpu.VMEM((rows,), jnp.float32),
                  pltpu.VMEM((_G,), jnp.float32),
                  pltpu.VMEM((2, D), jnp.float32),
                  pltpu.VMEM_SHARED((_R + 16, D), jnp.float32),
                  pltpu.VMEM_SHARED((_R + 16,), jnp.float32)])
    s_pad, cnt_pad, st32 = pl.kernel(
        body, out_type=out_type, mesh=mesh, scratch_types=scratch,
        compiler_params=pltpu.CompilerParams(needs_layout_passes=False),
    )(ptab, qtab, srcp, shp, dstp)
    return s_pad, cnt_pad, st32


def _pad_edges(src, sh, dst):
    e = src.shape[0]
    e_pad = -(-e // _EALIGN) * _EALIGN
    pad = e_pad - e
    srcp = jnp.concatenate([src, jnp.zeros((pad,), jnp.int32)])
    shp = jnp.concatenate([sh, jnp.zeros((pad,), jnp.int32)])
    dstp = jnp.concatenate([dst, jnp.full((pad,), 2 ** 30, jnp.int32)])
    return srcp, shp, dstp


def _bn_affine(ssum, ssq, e, g, be):
    m = ssum / e
    v = ssq / e - m * m
    a = g * lax.rsqrt(v + 1e-5)
    c = be - a * m
    return a, c


def _aff8(ax, cx, au, cu, ad, cd, b1, b2):
    return jnp.stack([ax, cx, au, cu, ad, cd, b1, b2], axis=0)


def kernel(x0, x1, x2, up_index0, up_shared0, up_index1, up_shared1,
           down_index1, down_shared1, down_index2, down_shared2,
           batch0, batch1, batch2, Wup, bup, gup, beup, Wdn, bdn, gdn, bedn,
           Wu1, bu1, Wu2, bu2, gu, beu, Wl1, bl1, Wl2, bl2):
    n0, n1, n2 = x0.shape[0], x1.shape[0], x2.shape[0]
    l_layers = Wup.shape[0]
    blk = 2000

    zer = jnp.zeros((D,), jnp.float32)
    one = jnp.ones((D,), jnp.float32)
    an0 = an1 = an2 = one
    cn0 = cn1 = cn2 = zer
    h0, h1, h2x = x0, x1, x2

    eu0 = _pad_edges(up_index0[0], up_shared0, up_index0[1])
    eu1 = _pad_edges(up_index1[0], up_shared1, up_index1[1])
    ed1 = _pad_edges(down_index1[0], down_shared1, down_index1[1])
    ed2 = _pad_edges(down_index2[0], down_shared2, down_index2[1])

    for l in range(l_layers):
        wu_t, wu_b = Wup[l][:D], Wup[l][D:]
        wd_t, wd_b = Wdn[l][:D], Wdn[l][D:]
        w0 = jnp.concatenate([wu_t, wd_b], axis=1)
        r0 = (cn0 @ w0 + jnp.concatenate([bup[l], jnp.zeros((D,))]))[None]
        t0 = _premult(h0, an0[:, None] * w0, r0, blk)

        w1c = jnp.concatenate([wu_b, wu_t, wd_t, wd_b], axis=1)
        r1 = (cn1 @ w1c + jnp.concatenate(
            [jnp.zeros((D,)), bup[l], bdn[l], jnp.zeros((D,))]))[None]
        t1 = _premult(h1, an1[:, None] * w1c, r1, blk)

        w2c = jnp.concatenate([wu_b, wd_t], axis=1)
        r2 = (cn2 @ w2c + jnp.concatenate([jnp.zeros((D,)), bdn[l]]))[None]
        t2 = _premult(h2x, an2[:, None] * w2c, r2, blk)

        s_u0, c_u0, st32 = _edge_sc(t0[0], t1[0], *eu0, n0)
        a_u0, cc_u0 = _bn_affine(st32[:, 0].sum(0), st32[:, 1].sum(0),
                                 up_shared0.shape[0], gup[l], beup[l])
        s_u1, c_u1, st32 = _edge_sc(t1[1], t2[0], *eu1, n1)
        a_u1, cc_u1 = _bn_affine(st32[:, 0].sum(0), st32[:, 1].sum(0),
                                 up_shared1.shape[0], gup[l], beup[l])
        s_d1, c_d1, st32 = _edge_sc(t1[2], t0[1], *ed1, n1)
        a_d1, cc_d1 = _bn_affine(st32[:, 0].sum(0), st32[:, 1].sum(0),
                                 down_shared1.shape[0], gdn[l], bedn[l])
        s_d2, c_d2, st32 = _edge_sc(t2[1], t1[3], *ed2, n2)
        a_d2, cc_d2 = _bn_affine(st32[:, 0].sum(0), st32[:, 1].sum(0),
                                 down_shared2.shape[0], gdn[l], bedn[l])

        h0, st0 = _update(h0, s_u0, c_u0[:, None], None, None,
                          _aff8(an0, cn0, a_u0, cc_u0, zer, zer, bu1[l], bu2[l]),
                          Wu1[l], Wu2[l], blk)
        h1, st1 = _update(h1, s_u1, c_u1[:, None], s_d1, c_d1[:, None],
                          _aff8(an1, cn1, a_u1, cc_u1, a_d1, cc_d1, bu1[l], bu2[l]),
                          Wu1[l], Wu2[l], blk)
        h2x, st2 = _update(h2x, s_d2, c_d2[:, None], None, None,
                           _aff8(an2, cn2, a_d2, cc_d2, zer, zer, bu1[l], bu2[l]),
                           Wu1[l], Wu2[l], blk)

        an0, cn0 = _bn_affine(st0[0], st0[1], float(n0), gu[l], beu[l])
        an1, cn1 = _bn_affine(st1[0], st1[1], float(n1), gu[l], beu[l])
        an2, cn2 = _bn_affine(st2[0], st2[1], float(n2), gu[l], beu[l])

    ps0, pc0 = _pool(h0, batch0, _aff8(an0, cn0, zer, zer, zer, zer, zer, zer), blk)
    ps1, pc1 = _pool(h1, batch1, _aff8(an1, cn1, zer, zer, zer, zer, zer, zer), blk)
    ps2, pc2 = _pool(h2x, batch2, _aff8(an2, cn2, zer, zer, zer, zer, zer, zer), blk)

    return _final(ps0, pc0, ps1, pc1, ps2, pc2, Wl1, bl1, Wl2, bl2)


# speedup vs baseline: 6.7602x; 1.2512x over previous
import functools

import jax
import jax.numpy as jnp
from jax import lax
from jax.experimental import pallas as pl
from jax.experimental.pallas import tpu as pltpu
from jax.experimental.pallas import tpu_sc as plsc

D = 128
NB = 32


def _premult_body(nt, x_ref, w_ref, r_ref, *o_refs):
    res = jnp.dot(x_ref[...], w_ref[...], preferred_element_type=jnp.float32)
    res = res + r_ref[...]
    for j in range(nt):
        o_refs[j][...] = res[:, j * D:(j + 1) * D]


def _premult(x, w, rowc, blk):
    n = x.shape[0]
    k = w.shape[1]
    nt = k // D
    return pl.pallas_call(
        functools.partial(_premult_body, nt),
        grid=(n // blk,),
        in_specs=[
            pl.BlockSpec((blk, D), lambda i: (i, 0)),
            pl.BlockSpec((D, k), lambda i: (0, 0)),
            pl.BlockSpec((1, k), lambda i: (0, 0)),
        ],
        out_specs=[pl.BlockSpec((blk, D), lambda i: (i, 0))] * nt,
        out_shape=[jax.ShapeDtypeStruct((n, D), jnp.float32)] * nt,
    )(x, w, rowc)


def _upd_body2(x_ref, su_ref, cu_ref, sd_ref, cd_ref, aff_ref, w1_ref, w2_ref,
               h_ref, st_ref):
    i = pl.program_id(0)
    aff = aff_ref[...]
    u = (x_ref[...] * aff[0] + aff[1]
         + su_ref[...] * aff[2] + cu_ref[...] * aff[3]
         + sd_ref[...] * aff[4] + cd_ref[...] * aff[5])
    h1 = jnp.maximum(jnp.dot(u, w1_ref[...], preferred_element_type=jnp.float32) + aff[6], 0.0)
    h2 = jnp.maximum(jnp.dot(h1, w2_ref[...], preferred_element_type=jnp.float32) + aff[7], 0.0)
    h_ref[...] = h2

    @pl.when(i == 0)
    def _():
        st_ref[...] = jnp.zeros_like(st_ref)

    s1 = jnp.sum(h2, axis=0)
    s2 = jnp.sum(h2 * h2, axis=0)
    st_ref[...] += jnp.concatenate(
        [s1[None], s2[None], jnp.zeros((6, D), jnp.float32)], axis=0)


def _upd_body1(x_ref, su_ref, cu_ref, aff_ref, w1_ref, w2_ref, h_ref, st_ref):
    i = pl.program_id(0)
    aff = aff_ref[...]
    u = (x_ref[...] * aff[0] + aff[1]
         + su_ref[...] * aff[2] + cu_ref[...] * aff[3])
    h1 = jnp.maximum(jnp.dot(u, w1_ref[...], preferred_element_type=jnp.float32) + aff[6], 0.0)
    h2 = jnp.maximum(jnp.dot(h1, w2_ref[...], preferred_element_type=jnp.float32) + aff[7], 0.0)
    h_ref[...] = h2

    @pl.when(i == 0)
    def _():
        st_ref[...] = jnp.zeros_like(st_ref)

    s1 = jnp.sum(h2, axis=0)
    s2 = jnp.sum(h2 * h2, axis=0)
    st_ref[...] += jnp.concatenate(
        [s1[None], s2[None], jnp.zeros((6, D), jnp.float32)], axis=0)


def _update(x, su, cu, sd, cd, aff, w1, w2, blk):
    n = x.shape[0]
    two = sd is not None
    body = _upd_body2 if two else _upd_body1
    in_arrays = [x, su, cu] + ([sd, cd] if two else []) + [aff, w1, w2]
    in_specs = ([pl.BlockSpec((blk, D), lambda i: (i, 0)),
                 pl.BlockSpec((blk, D), lambda i: (i, 0)),
                 pl.BlockSpec((blk, 1), lambda i: (i, 0))]
                + ([pl.BlockSpec((blk, D), lambda i: (i, 0)),
                    pl.BlockSpec((blk, 1), lambda i: (i, 0))] if two else [])
                + [pl.BlockSpec((8, D), lambda i: (0, 0)),
                   pl.BlockSpec((D, D), lambda i: (0, 0)),
                   pl.BlockSpec((D, D), lambda i: (0, 0))])
    return pl.pallas_call(
        body,
        grid=(n // blk,),
        in_specs=in_specs,
        out_specs=[pl.BlockSpec((blk, D), lambda i: (i, 0)),
                   pl.BlockSpec((8, D), lambda i: (0, 0))],
        out_shape=[jax.ShapeDtypeStruct((n, D), jnp.float32),
                   jax.ShapeDtypeStruct((8, D), jnp.float32)],
    )(*in_arrays)


def _pool_body(h_ref, b_ref, aff_ref, ps_ref, pc_ref):
    i = pl.program_id(0)

    @pl.when(i == 0)
    def _():
        ps_ref[...] = jnp.zeros_like(ps_ref)
        pc_ref[...] = jnp.zeros_like(pc_ref)

    aff = aff_ref[...]
    xnew = h_ref[...] * aff[0] + aff[1]
    bid = b_ref[...]
    cols = lax.broadcasted_iota(jnp.int32, (bid.shape[0], NB), 1)
    onehot = (bid == cols).astype(jnp.float32)
    ps_ref[...] += lax.dot_general(onehot, xnew, (((0,), (0,)), ((), ())),
                                   preferred_element_type=jnp.float32)
    pc_ref[...] += jnp.sum(onehot, axis=0)[:, None]


def _pool(h, batch, aff, blk):
    n = h.shape[0]
    return pl.pallas_call(
        _pool_body,
        grid=(n // blk,),
        in_specs=[pl.BlockSpec((blk, D), lambda i: (i, 0)),
                  pl.BlockSpec((blk, 1), lambda i: (i, 0)),
                  pl.BlockSpec((8, D), lambda i: (0, 0))],
        out_specs=[pl.BlockSpec((NB, D), lambda i: (0, 0)),
                   pl.BlockSpec((NB, D), lambda i: (0, 0))],
        out_shape=[jax.ShapeDtypeStruct((NB, D), jnp.float32),
                   jax.ShapeDtypeStruct((NB, D), jnp.float32)],
    )(h, batch.reshape(n, 1), aff)


def _final_body(ps0, pc0, ps1, pc1, ps2, pc2, w1_ref, b1_ref, w2_ref, b2_ref,
                o_ref):
    p = (ps0[...] / jnp.maximum(pc0[...], 1.0)
         + ps1[...] / jnp.maximum(pc1[...], 1.0)
         + ps2[...] / jnp.maximum(pc2[...], 1.0))
    h = jnp.maximum(jnp.dot(p, w1_ref[...], preferred_element_type=jnp.float32)
                    + b1_ref[...], 0.0)
    o = jnp.dot(h, w2_ref[...], preferred_element_type=jnp.float32) + b2_ref[...]
    m = jnp.max(o, axis=-1, keepdims=True)
    lse = m + jnp.log(jnp.sum(jnp.exp(o - m), axis=-1, keepdims=True))
    o_ref[...] = o - lse


def _final(ps0, pc0, ps1, pc1, ps2, pc2, wl1, bl1, wl2, bl2):
    c = wl2.shape[1]
    return pl.pallas_call(
        _final_body,
        out_shape=jax.ShapeDtypeStruct((NB, c), jnp.float32),
    )(ps0, pc0, ps1, pc1, ps2, pc2, wl1, bl1.reshape(1, -1), wl2,
      bl2.reshape(1, -1))


_R = 10240
_CE = 1024
_G = 128
_CB = 1168
_EALIGN = 2 * 16 * _CE


def _edge_sc(ptab, qtab, idx3, ndst):
    e_pad = idx3.shape[0] // 3
    nbin = -(-ndst // _R)
    nch = e_pad // (16 * _CE)
    npair = nch // 2
    rows = _R // 16

    def body(ptab_h, qtab_h, idx_h, s_h, cnt_h, st_h,
             sidx0, sidx1, csrc, csh, cdst, gsrc, gsh, gdst,
             pbuf, zbuf, zcnt, onesg, stv, acc, cntacc, sem0, sem1):
        c = lax.axis_index("c")
        s = lax.axis_index("s")
        iota = lax.iota(jnp.int32, 16)
        zv = jnp.zeros((16,), jnp.float32)

        def start_stage(ch, buf, sem):
            g = s * nch + jnp.minimum(ch, nch - 1)
            return pltpu.async_copy(idx_h.at[pl.ds(g * (3 * _CE), 3 * _CE)],
                                    buf, sem)

        def z16(r, _):
            for kc in range(8):
                zbuf[r, pl.ds(kc * 16, 16)] = zv
            return 0
        lax.fori_loop(0, 128, z16, 0)

        def zc16(k, _):
            zcnt[pl.ds(k * 16, 16)] = zv
            return 0
        lax.fori_loop(0, rows // 16, zc16, 0)

        def o16(k, _):
            onesg[pl.ds(k * 16, 16)] = zv + 1.0
            return 0
        lax.fori_loop(0, _G // 16, o16, 0)
        for kc in range(8):
            stv[0, pl.ds(kc * 16, 16)] = zv
            stv[1, pl.ds(kc * 16, 16)] = zv

        def process_group(goff, nvalid, full):
            def stage(k, _):
                base = goff + k * 16
                sv = csrc[pl.ds(base, 16)]
                hv = csh[pl.ds(base, 16)]
                dv = cdst[pl.ds(base, 16)]
                if not full:
                    slot = k * 16 + iota
                    valid = slot < nvalid
                    sv = jnp.where(valid, sv, slot & 127)
                    hv = jnp.where(valid, hv, slot & 127)
                    dv = jnp.where(valid, dv, _R + (iota & 15))
                gsrc[pl.ds(k * 16, 16)] = sv
                gsh[pl.ds(k * 16, 16)] = hv
                gdst[pl.ds(k * 16, 16)] = dv
                return 0
            lax.fori_loop(0, _G // 16, stage, 0)
            pltpu.sync_copy(ptab_h.at[gsrc], pbuf)
            pltpu.sync_copy(qtab_h.at[gsh], pbuf, add=True)

            def crow(r, st):
                out = []
                for kc in range(8):
                    y = jnp.maximum(pbuf[r, pl.ds(kc * 16, 16)], 0.0)
                    pbuf[r, pl.ds(kc * 16, 16)] = y
                    if not full:
                        y = jnp.where(r < nvalid, y, 0.0)
                    out.append((st[kc][0] + y, st[kc][1] + y * y))
                return tuple(out)
            st0 = tuple((zv, zv) for _ in range(8))
            st = lax.fori_loop(0, _G, crow, st0)
            for kc in range(8):
                stv[0, pl.ds(kc * 16, 16)] += st[kc][0]
                stv[1, pl.ds(kc * 16, 16)] += st[kc][1]
            pltpu.sync_copy(pbuf, acc.at[gdst], add=True)
            pltpu.sync_copy(onesg, cntacc.at[gdst], add=True)

        def wait_stage(buf, sem):
            pltpu.make_async_copy(idx_h.at[pl.ds(0, 3 * _CE)], buf, sem).wait()

        def do_bin(i, _):
            b = 2 * i + c
            lo = b * _R
            cps = [pltpu.async_copy(
                zbuf, acc.at[pl.ds(s * rows + j * 64, 64)], sem0)
                for j in range(rows // 64)]
            pltpu.sync_copy(zcnt, cntacc.at[pl.ds(s * rows, rows)])
            for cp in cps:
                cp.wait()
            plsc.subcore_barrier()

            def do_chunk_from(buf, cur):
                def filt(j, cur):
                    dv = buf[pl.ds(2 * _CE + j * 16, 16)]
                    m = (dv >= lo) & (dv < lo + _R)
                    npass = jnp.max(plsc.all_reduce_population_count(m))

                    @pl.when(npass > 0)
                    def _():
                        csum = plsc.cumsum(m.astype(jnp.int32))
                        pos = jnp.where(m, cur + csum - 1, _CB + iota)
                        plsc.store_scatter(cdst, [pos], dv - lo)
                        plsc.store_scatter(csrc, [pos],
                                           buf[pl.ds(j * 16, 16)])
                        plsc.store_scatter(csh, [pos],
                                           buf[pl.ds(_CE + j * 16, 16)])
                    return cur + npass
                cur = lax.fori_loop(0, _CE // 16, filt, cur)
                ngr = cur // _G

                def dog(g, _):
                    process_group(g * _G, _G, True)
                    return 0
                lax.fori_loop(0, ngr, dog, 0)

                def shift(k, _):
                    v1 = csrc[pl.ds(ngr * _G + k * 16, 16)]
                    v2 = csh[pl.ds(ngr * _G + k * 16, 16)]
                    v3 = cdst[pl.ds(ngr * _G + k * 16, 16)]
                    csrc[pl.ds(k * 16, 16)] = v1
                    csh[pl.ds(k * 16, 16)] = v2
                    cdst[pl.ds(k * 16, 16)] = v3
                    return 0
                lax.fori_loop(0, _G // 16, shift, 0)
                return cur - ngr * _G

            start_stage(jnp.int32(0), sidx0, sem0)

            def do_pair(ip, cur):
                start_stage(2 * ip + 1, sidx1, sem1)
                wait_stage(sidx0, sem0)
                cur = do_chunk_from(sidx0, cur)
                start_stage(2 * ip + 2, sidx0, sem0)
                wait_stage(sidx1, sem1)
                cur = do_chunk_from(sidx1, cur)
                return cur

            rem = lax.fori_loop(0, npair, do_pair, jnp.int32(0))
            wait_stage(sidx0, sem0)

            @pl.when(rem > 0)
            def _():
                process_group(0, rem, False)

            plsc.subcore_barrier()
            pltpu.sync_copy(acc.at[pl.ds(s * rows, rows)],
                            s_h.at[pl.ds(b * _R + s * rows, rows)])
            pltpu.sync_copy(cntacc.at[pl.ds(s * rows, rows)],
                            cnt_h.at[pl.ds(b * _R + s * rows, rows)])
            return 0

        nb_me = jnp.where(c == 0, (nbin + 1) // 2, nbin // 2)
        lax.fori_loop(0, nb_me, do_bin, 0)
        pltpu.sync_copy(stv, st_h.at[s * 2 + c])

    mesh = plsc.VectorSubcoreMesh(core_axis_name="c", subcore_axis_name="s",
                                  num_cores=2, num_subcores=16)
    out_type = [jax.ShapeDtypeStruct((nbin * _R, D), jnp.float32),
                jax.ShapeDtypeStruct((nbin * _R,), jnp.float32),
                jax.ShapeDtypeStruct((32, 2, D), jnp.float32)]
    scratch = ([pltpu.VMEM((3 * _CE,), jnp.int32)] * 2
               + [pltpu.VMEM((_CB + 16,), jnp.int32)] * 3
               + [pltpu.VMEM((_G,), jnp.int32)] * 3
               + [pltpu.VMEM((_G, D), jnp.float32),
                  pltpu.VMEM((64, D), jnp.float32),
                  pltpu.VMEM((rows,), jnp.float32),
                  pltpu.VMEM((_G,), jnp.float32),
                  pltpu.VMEM((2, D), jnp.float32),
                  pltpu.VMEM_SHARED((_R + 16, D), jnp.float32),
                  pltpu.VMEM_SHARED((_R + 16,), jnp.float32),
                  pltpu.SemaphoreType.DMA,
                  pltpu.SemaphoreType.DMA])
    s_pad, cnt_pad, st32 = pl.kernel(
        body, out_type=out_type, mesh=mesh, scratch_types=scratch,
        compiler_params=pltpu.CompilerParams(
            needs_layout_passes=False, internal_scratch_in_bytes=65536),
    )(ptab, qtab, idx3)
    return s_pad, cnt_pad, st32


def _pad_edges(src, sh, dst):
    e = src.shape[0]
    e_pad = -(-e // _EALIGN) * _EALIGN
    pad = e_pad - e
    srcp = jnp.concatenate([src, jnp.zeros((pad,), jnp.int32)])
    shp = jnp.concatenate([sh, jnp.zeros((pad,), jnp.int32)])
    dstp = jnp.concatenate([dst, jnp.full((pad,), 2 ** 30, jnp.int32)])
    nch_tot = e_pad // _CE
    idx3 = jnp.stack([srcp.reshape(nch_tot, _CE), shp.reshape(nch_tot, _CE),
                      dstp.reshape(nch_tot, _CE)], axis=1).reshape(-1)
    return (idx3,)


def _bn_affine(ssum, ssq, e, g, be):
    m = ssum / e
    v = ssq / e - m * m
    a = g * lax.rsqrt(v + 1e-5)
    c = be - a * m
    return a, c


def _aff8(ax, cx, au, cu, ad, cd, b1, b2):
    return jnp.stack([ax, cx, au, cu, ad, cd, b1, b2], axis=0)


def kernel(x0, x1, x2, up_index0, up_shared0, up_index1, up_shared1,
           down_index1, down_shared1, down_index2, down_shared2,
           batch0, batch1, batch2, Wup, bup, gup, beup, Wdn, bdn, gdn, bedn,
           Wu1, bu1, Wu2, bu2, gu, beu, Wl1, bl1, Wl2, bl2):
    n0, n1, n2 = x0.shape[0], x1.shape[0], x2.shape[0]
    l_layers = Wup.shape[0]
    blk = 2000

    zer = jnp.zeros((D,), jnp.float32)
    one = jnp.ones((D,), jnp.float32)
    an0 = an1 = an2 = one
    cn0 = cn1 = cn2 = zer
    h0, h1, h2x = x0, x1, x2

    eu0 = _pad_edges(up_index0[0], up_shared0, up_index0[1])
    eu1 = _pad_edges(up_index1[0], up_shared1, up_index1[1])
    ed1 = _pad_edges(down_index1[0], down_shared1, down_index1[1])
    ed2 = _pad_edges(down_index2[0], down_shared2, down_index2[1])

    for l in range(l_layers):
        wu_t, wu_b = Wup[l][:D], Wup[l][D:]
        wd_t, wd_b = Wdn[l][:D], Wdn[l][D:]
        w0 = jnp.concatenate([wu_t, wd_b], axis=1)
        r0 = (cn0 @ w0 + jnp.concatenate([bup[l], jnp.zeros((D,))]))[None]
        t0 = _premult(h0, an0[:, None] * w0, r0, blk)

        w1c = jnp.concatenate([wu_b, wu_t, wd_t, wd_b], axis=1)
        r1 = (cn1 @ w1c + jnp.concatenate(
            [jnp.zeros((D,)), bup[l], bdn[l], jnp.zeros((D,))]))[None]
        t1 = _premult(h1, an1[:, None] * w1c, r1, blk)

        w2c = jnp.concatenate([wu_b, wd_t], axis=1)
        r2 = (cn2 @ w2c + jnp.concatenate([jnp.zeros((D,)), bdn[l]]))[None]
        t2 = _premult(h2x, an2[:, None] * w2c, r2, blk)

        s_u0, c_u0, st32 = _edge_sc(t0[0], t1[0], *eu0, n0)
        a_u0, cc_u0 = _bn_affine(st32[:, 0].sum(0), st32[:, 1].sum(0),
                                 up_shared0.shape[0], gup[l], beup[l])
        s_u1, c_u1, st32 = _edge_sc(t1[1], t2[0], *eu1, n1)
        a_u1, cc_u1 = _bn_affine(st32[:, 0].sum(0), st32[:, 1].sum(0),
                                 up_shared1.shape[0], gup[l], beup[l])
        s_d1, c_d1, st32 = _edge_sc(t1[2], t0[1], *ed1, n1)
        a_d1, cc_d1 = _bn_affine(st32[:, 0].sum(0), st32[:, 1].sum(0),
                                 down_shared1.shape[0], gdn[l], bedn[l])
        s_d2, c_d2, st32 = _edge_sc(t2[1], t1[3], *ed2, n2)
        a_d2, cc_d2 = _bn_affine(st32[:, 0].sum(0), st32[:, 1].sum(0),
                                 down_shared2.shape[0], gdn[l], bedn[l])

        h0, st0 = _update(h0, s_u0, c_u0[:, None], None, None,
                          _aff8(an0, cn0, a_u0, cc_u0, zer, zer, bu1[l], bu2[l]),
                          Wu1[l], Wu2[l], blk)
        h1, st1 = _update(h1, s_u1, c_u1[:, None], s_d1, c_d1[:, None],
                          _aff8(an1, cn1, a_u1, cc_u1, a_d1, cc_d1, bu1[l], bu2[l]),
                          Wu1[l], Wu2[l], blk)
        h2x, st2 = _update(h2x, s_d2, c_d2[:, None], None, None,
                           _aff8(an2, cn2, a_d2, cc_d2, zer, zer, bu1[l], bu2[l]),
                           Wu1[l], Wu2[l], blk)

        an0, cn0 = _bn_affine(st0[0], st0[1], float(n0), gu[l], beu[l])
        an1, cn1 = _bn_affine(st1[0], st1[1], float(n1), gu[l], beu[l])
        an2, cn2 = _bn_affine(st2[0], st2[1], float(n2), gu[l], beu[l])

    ps0, pc0 = _pool(h0, batch0, _aff8(an0, cn0, zer, zer, zer, zer, zer, zer), blk)
    ps1, pc1 = _pool(h1, batch1, _aff8(an1, cn1, zer, zer, zer, zer, zer, zer), blk)
    ps2, pc2 = _pool(h2x, batch2, _aff8(an2, cn2, zer, zer, zer, zer, zer, zer), blk)

    return _final(ps0, pc0, ps1, pc1, ps2, pc2, Wl1, bl1, Wl2, bl2)
